# baseline (speedup 1.0000x reference)
import functools

import jax
import jax.numpy as jnp
from jax import lax
from jax.experimental import pallas as pl
from jax.experimental.pallas import tpu as pltpu
from jax.experimental.pallas import tpu_sc as plsc

N = 10000
E = 320000
D = 128
C = 40
NCORES = 2
NSUB = 16
NTILES = NCORES * NSUB
CH = 128
NCH = 80
EPT = NCH * CH
NACC = 10112
RPT = NACC // NSUB
NPAD = NACC
DEGN = NSUB * 640
BLK = 1000
GRID = N // BLK
BLKP = NACC // 8
GRIDP = 8

_mesh = plsc.VectorSubcoreMesh(core_axis_name="c", subcore_axis_name="s")


def _deg_body(row_hbm, col_hbm, zdeg_hbm, deg0_out, deg1_out, roweff_out,
              row_v, col_v, eff_v, ones_v, deg_sp):
    c = lax.axis_index("c")
    s = lax.axis_index("s")
    wid = c * NSUB + s
    pltpu.sync_copy(zdeg_hbm.at[pl.ds(s * 640, 640)],
                    deg_sp.at[pl.ds(s * 640, 640)])
    for m in range(CH // 16):
        ones_v[pl.ds(m * 16, 16)] = jnp.ones((16,), jnp.float32)
    pltpu.sync_copy(row_hbm.at[wid], row_v)
    pltpu.sync_copy(col_hbm.at[wid], col_v)
    plsc.subcore_barrier()

    def chunk(j, carry):
        for m in range(CH // 16):
            r = row_v[j, pl.ds(m * 16, 16)]
            cc = col_v[j, pl.ds(m * 16, 16)]
            eff_v[j, pl.ds(m * 16, 16)] = jnp.where(r == cc, N, r)
        pltpu.sync_copy(ones_v, deg_sp.at[eff_v.at[j]], add=True)
        return carry

    lax.fori_loop(0, NCH, chunk, 0)
    pltpu.sync_copy(eff_v, roweff_out.at[wid])
    plsc.subcore_barrier()

    @pl.when(jnp.logical_and(s < 8, c == 0))
    def _():
        pltpu.sync_copy(deg_sp.at[pl.ds(s * 1280, 1280)],
                        deg0_out.at[pl.ds(s * 1280, 1280)])

    @pl.when(jnp.logical_and(s < 8, c == 1))
    def _():
        pltpu.sync_copy(deg_sp.at[pl.ds(s * 1280, 1280)],
                        deg1_out.at[pl.ds(s * 1280, 1280)])


@functools.partial(
    pl.kernel,
    out_type=(jax.ShapeDtypeStruct((DEGN,), jnp.float32),
              jax.ShapeDtypeStruct((DEGN,), jnp.float32),
              jax.ShapeDtypeStruct((NTILES, NCH, CH), jnp.int32)),
    mesh=_mesh,
    scratch_types=[
        pltpu.VMEM((NCH, CH), jnp.int32),
        pltpu.VMEM((NCH, CH), jnp.int32),
        pltpu.VMEM((NCH, CH), jnp.int32),
        pltpu.VMEM((CH,), jnp.float32),
        pltpu.VMEM_SHARED((DEGN,), jnp.float32),
    ],
)
def _sc_deg(row_hbm, col_hbm, zdeg_hbm, deg0_out, deg1_out, roweff_out,
            row_v, col_v, eff_v, ones_v, deg_sp):
    _deg_body(row_hbm, col_hbm, zdeg_hbm, deg0_out, deg1_out, roweff_out,
              row_v, col_v, eff_v, ones_v, deg_sp)


DH = D // 2


NBUF = 8
PFD = 6


def _scat_body(reff_hbm, col_hbm, u_hbm, z_hbm, out_hbm,
               reff_v, col_v, bufs, gsems, ssems, acc_sp):
    c = lax.axis_index("c")
    s = lax.axis_index("s")
    wid = c * NSUB + s
    pltpu.sync_copy(reff_hbm.at[wid], reff_v)
    pltpu.sync_copy(col_hbm.at[wid], col_v)

    for ph in range(2):
        pltpu.sync_copy(z_hbm.at[pl.ds(s * RPT, RPT)],
                        acc_sp.at[pl.ds(s * RPT, RPT)])
        plsc.subcore_barrier()
        u_ph = u_hbm.at[ph]

        def start_gather(j, b):
            pltpu.make_async_copy(u_ph.at[reff_v.at[j]], bufs[b],
                                  gsems[b]).start()

        def visit(j, b, prefetch):
            pltpu.make_async_copy(u_ph.at[reff_v.at[j]], bufs[b],
                                  gsems[b]).wait()
            if prefetch:
                start_gather(j + PFD, (b + PFD) % NBUF)
            pltpu.sync_copy(bufs[b], acc_sp.at[col_v.at[j]], add=True)

        for b in range(PFD):
            start_gather(b, b)

        def group(i, carry):
            for b in range(NBUF):
                j = i * NBUF + b
                visit(j, b, True)
            return carry

        def tail(_, carry):
            base = NCH - NBUF
            for b in range(NBUF):
                j = base + b
                visit(j, b, b + PFD < NBUF)
            return carry

        lax.fori_loop(0, NCH // NBUF - 1, group, 0)
        tail(0, 0)
        plsc.subcore_barrier()
        pltpu.sync_copy(acc_sp.at[pl.ds(s * RPT, RPT)],
                        out_hbm.at[c, ph, pl.ds(s * RPT, RPT)])


@functools.partial(
    pl.kernel,
    out_type=jax.ShapeDtypeStruct((NCORES, 2, NACC, DH), jnp.float32),
    mesh=_mesh,
    scratch_types=[
        pltpu.VMEM((NCH, CH), jnp.int32),
        pltpu.VMEM((NCH, CH), jnp.int32),
        [pltpu.VMEM((CH, DH), jnp.float32) for _ in range(NBUF)],
        [pltpu.SemaphoreType.DMA for _ in range(NBUF)],
        [pltpu.SemaphoreType.DMA for _ in range(NBUF)],
        pltpu.VMEM_SHARED((NACC, DH), jnp.float32),
    ],
    compiler_params=pltpu.CompilerParams(use_tc_tiling_on_sc=False),
)
def _sc_scatter(reff_hbm, col_hbm, u_hbm, z_hbm, out_hbm,
                reff_v, col_v, bufs, gsems, ssems, acc_sp):
    _scat_body(reff_hbm, col_hbm, u_hbm, z_hbm, out_hbm,
               reff_v, col_v, bufs, gsems, ssems, acc_sp)


def _tc_a_body(x_ref, w_ref, b_ref, d0_ref, d1_ref, h_ref, u_ref, dis_ref):
    h = jnp.tanh(jnp.dot(x_ref[...], w_ref[...],
                         preferred_element_type=jnp.float32) + b_ref[...])
    d = d0_ref[...] + d1_ref[...]
    dis = jnp.where(d > 0, lax.rsqrt(jnp.maximum(d, 1e-12)), 0.0)
    h_ref[...] = h
    dis_ref[...] = dis
    u = dis * h
    u_ref[0] = u[:, :DH]
    u_ref[1] = u[:, DH:]


def _tc_a(x, W_in, b_in, deg0, deg1):
    return pl.pallas_call(
        _tc_a_body,
        grid=(GRIDP,),
        in_specs=[
            pl.BlockSpec((BLKP, D), lambda i: (i, 0)),
            pl.BlockSpec((D, D), lambda i: (0, 0)),
            pl.BlockSpec((1, D), lambda i: (0, 0)),
            pl.BlockSpec((BLKP, 1), lambda i: (i, 0)),
            pl.BlockSpec((BLKP, 1), lambda i: (i, 0)),
        ],
        out_specs=[
            pl.BlockSpec((BLKP, D), lambda i: (i, 0)),
            pl.BlockSpec((2, BLKP, DH), lambda i: (0, i, 0)),
            pl.BlockSpec((BLKP, 1), lambda i: (i, 0)),
        ],
        out_shape=[
            jax.ShapeDtypeStruct((NACC, D), jnp.float32),
            jax.ShapeDtypeStruct((2, NACC, DH), jnp.float32),
            jax.ShapeDtypeStruct((NACC, 1), jnp.float32),
        ],
    )(x, W_in, b_in, deg0, deg1)


def _tc_b_body(s_ref, dis_ref, tx1_ref, u2_ref):
    dis = dis_ref[...]
    ss = jnp.concatenate([s_ref[0, 0] + s_ref[1, 0],
                          s_ref[0, 1] + s_ref[1, 1]], axis=1)
    tx1 = -dis * ss
    tx1_ref[...] = tx1
    u2 = dis * tx1
    u2_ref[0] = u2[:, :DH]
    u2_ref[1] = u2[:, DH:]


def _tc_b(s1, dis):
    return pl.pallas_call(
        _tc_b_body,
        grid=(GRIDP,),
        in_specs=[
            pl.BlockSpec((NCORES, 2, BLKP, DH), lambda i: (0, 0, i, 0)),
            pl.BlockSpec((BLKP, 1), lambda i: (i, 0)),
        ],
        out_specs=[
            pl.BlockSpec((BLKP, D), lambda i: (i, 0)),
            pl.BlockSpec((2, BLKP, DH), lambda i: (0, i, 0)),
        ],
        out_shape=[
            jax.ShapeDtypeStruct((NACC, D), jnp.float32),
            jax.ShapeDtypeStruct((2, NACC, DH), jnp.float32),
        ],
    )(s1, dis)


def _tc_c_body(h_ref, tx1_ref, s_ref, dis_ref, wc_ref, bc_ref, wo_ref,
               bo_ref, y_ref):
    h = h_ref[...]
    tx1 = tx1_ref[...]
    ss = jnp.concatenate([s_ref[0, 0] + s_ref[1, 0],
                          s_ref[0, 1] + s_ref[1, 1]], axis=1)
    tx2 = -2.0 * dis_ref[...] * ss - h
    out = (jnp.dot(h, wc_ref[0], preferred_element_type=jnp.float32)
           + jnp.dot(tx1, wc_ref[1], preferred_element_type=jnp.float32)
           + jnp.dot(tx2, wc_ref[2], preferred_element_type=jnp.float32)
           + bc_ref[...])
    y_ref[...] = jnp.dot(jnp.maximum(out, 0.0), wo_ref[...],
                         preferred_element_type=jnp.float32) + bo_ref[...]


def _tc_c(h, tx1, s2, dis, W_cheb, b_cheb, W_out, b_out):
    return pl.pallas_call(
        _tc_c_body,
        grid=(GRID,),
        in_specs=[
            pl.BlockSpec((BLK, D), lambda i: (i, 0)),
            pl.BlockSpec((BLK, D), lambda i: (i, 0)),
            pl.BlockSpec((NCORES, 2, BLK, DH), lambda i: (0, 0, i, 0)),
            pl.BlockSpec((BLK, 1), lambda i: (i, 0)),
            pl.BlockSpec((3, D, D), lambda i: (0, 0, 0)),
            pl.BlockSpec((1, D), lambda i: (0, 0)),
            pl.BlockSpec((D, C), lambda i: (0, 0)),
            pl.BlockSpec((1, C), lambda i: (0, 0)),
        ],
        out_specs=pl.BlockSpec((BLK, C), lambda i: (i, 0)),
        out_shape=jax.ShapeDtypeStruct((N, C), jnp.float32),
    )(h, tx1, s2, dis, W_cheb, b_cheb, W_out, b_out)


def kernel(x, edge_index, W_in, b_in, W_cheb, b_cheb, W_out, b_out):
    npad_e = NTILES * EPT - E
    k = jnp.arange(npad_e, dtype=jnp.int32)
    epad = jnp.stack([N + (k + 50) % (NACC - N), N + k % (NACC - N)])
    ei = jnp.concatenate([edge_index, epad], axis=1)
    row = ei[0].reshape(NTILES, NCH, CH)
    col = ei[1].reshape(NTILES, NCH, CH)
    zdeg = jnp.zeros((DEGN,), jnp.float32)
    znd = jnp.zeros((NACC, DH), jnp.float32)

    xp = jnp.concatenate([x, jnp.zeros((NACC - N, D), jnp.float32)], axis=0)
    deg0, deg1, roweff = _sc_deg(row, col, zdeg)
    h, u1, dis = _tc_a(xp, W_in, b_in.reshape(1, D),
                       deg0[:NACC].reshape(NACC, 1),
                       deg1[:NACC].reshape(NACC, 1))
    s1 = _sc_scatter(roweff, col, u1, znd)
    tx1, u2 = _tc_b(s1, dis)
    s2 = _sc_scatter(roweff, col, u2, znd)
    return _tc_c(h, tx1, s2, dis, W_cheb, b_cheb.reshape(1, D), W_out,
                 b_out.reshape(1, C))


# speedup vs baseline: 1.0019x; 1.0019x over previous
import functools

import jax
import jax.numpy as jnp
from jax import lax
from jax.experimental import pallas as pl
from jax.experimental.pallas import tpu as pltpu
from jax.experimental.pallas import tpu_sc as plsc

N = 10000
E = 320000
D = 128
C = 40
NCORES = 2
NSUB = 16
NTILES = NCORES * NSUB
CH = 128
NCH = 80
EPT = NCH * CH
NACC = 10112
RPT = NACC // NSUB
NPAD = NACC
DEGN = NSUB * 640
BLK = 1000
GRID = N // BLK
BLKP = NACC // 8
GRIDP = 8

_mesh = plsc.VectorSubcoreMesh(core_axis_name="c", subcore_axis_name="s")


def _deg_body(row_hbm, col_hbm, zdeg_hbm, deg0_out, deg1_out, roweff_out,
              row_v, col_v, eff_v, ones_v, deg_sp):
    c = lax.axis_index("c")
    s = lax.axis_index("s")
    wid = c * NSUB + s
    pltpu.sync_copy(zdeg_hbm.at[pl.ds(s * 640, 640)],
                    deg_sp.at[pl.ds(s * 640, 640)])
    for m in range(CH // 16):
        ones_v[pl.ds(m * 16, 16)] = jnp.ones((16,), jnp.float32)
    pltpu.sync_copy(row_hbm.at[wid], row_v)
    pltpu.sync_copy(col_hbm.at[wid], col_v)
    plsc.subcore_barrier()

    def chunk(j, carry):
        for m in range(CH // 16):
            r = row_v[j, pl.ds(m * 16, 16)]
            cc = col_v[j, pl.ds(m * 16, 16)]
            eff_v[j, pl.ds(m * 16, 16)] = jnp.where(r == cc, N, r)
        pltpu.sync_copy(ones_v, deg_sp.at[eff_v.at[j]], add=True)
        return carry

    lax.fori_loop(0, NCH, chunk, 0)
    pltpu.sync_copy(eff_v, roweff_out.at[wid])
    plsc.subcore_barrier()

    @pl.when(jnp.logical_and(s < 8, c == 0))
    def _():
        pltpu.sync_copy(deg_sp.at[pl.ds(s * 1280, 1280)],
                        deg0_out.at[pl.ds(s * 1280, 1280)])

    @pl.when(jnp.logical_and(s < 8, c == 1))
    def _():
        pltpu.sync_copy(deg_sp.at[pl.ds(s * 1280, 1280)],
                        deg1_out.at[pl.ds(s * 1280, 1280)])


@functools.partial(
    pl.kernel,
    out_type=(jax.ShapeDtypeStruct((DEGN,), jnp.float32),
              jax.ShapeDtypeStruct((DEGN,), jnp.float32),
              jax.ShapeDtypeStruct((NTILES, NCH, CH), jnp.int32)),
    mesh=_mesh,
    scratch_types=[
        pltpu.VMEM((NCH, CH), jnp.int32),
        pltpu.VMEM((NCH, CH), jnp.int32),
        pltpu.VMEM((NCH, CH), jnp.int32),
        pltpu.VMEM((CH,), jnp.float32),
        pltpu.VMEM_SHARED((DEGN,), jnp.float32),
    ],
)
def _sc_deg(row_hbm, col_hbm, zdeg_hbm, deg0_out, deg1_out, roweff_out,
            row_v, col_v, eff_v, ones_v, deg_sp):
    _deg_body(row_hbm, col_hbm, zdeg_hbm, deg0_out, deg1_out, roweff_out,
              row_v, col_v, eff_v, ones_v, deg_sp)


DH = D // 2


NBUF = 8
PFD = 4


def _scat_body(reff_hbm, col_hbm, u_hbm, z_hbm, out_hbm,
               reff_v, col_v, bufs, gsems, ssems, acc_sp):
    c = lax.axis_index("c")
    s = lax.axis_index("s")
    wid = c * NSUB + s
    pltpu.sync_copy(reff_hbm.at[wid], reff_v)
    pltpu.sync_copy(col_hbm.at[wid], col_v)

    for ph in range(2):
        pltpu.sync_copy(z_hbm.at[pl.ds(s * RPT, RPT)],
                        acc_sp.at[pl.ds(s * RPT, RPT)])
        plsc.subcore_barrier()
        u_ph = u_hbm.at[ph]

        def start_gather(j, b):
            pltpu.make_async_copy(u_ph.at[reff_v.at[j]], bufs[b],
                                  gsems[b]).start()

        def visit(j, b, prefetch):
            pltpu.make_async_copy(u_ph.at[reff_v.at[j]], bufs[b],
                                  gsems[b]).wait()
            if prefetch:
                start_gather(j + PFD, (b + PFD) % NBUF)
            pltpu.sync_copy(bufs[b], acc_sp.at[col_v.at[j]], add=True)

        for b in range(PFD):
            start_gather(b, b)

        def group(i, carry):
            for b in range(NBUF):
                j = i * NBUF + b
                visit(j, b, True)
            return carry

        def tail(_, carry):
            base = NCH - NBUF
            for b in range(NBUF):
                j = base + b
                visit(j, b, b + PFD < NBUF)
            return carry

        lax.fori_loop(0, NCH // NBUF - 1, group, 0)
        tail(0, 0)
        plsc.subcore_barrier()
        pltpu.sync_copy(acc_sp.at[pl.ds(s * RPT, RPT)],
                        out_hbm.at[c, ph, pl.ds(s * RPT, RPT)])


@functools.partial(
    pl.kernel,
    out_type=jax.ShapeDtypeStruct((NCORES, 2, NACC, DH), jnp.float32),
    mesh=_mesh,
    scratch_types=[
        pltpu.VMEM((NCH, CH), jnp.int32),
        pltpu.VMEM((NCH, CH), jnp.int32),
        [pltpu.VMEM((CH, DH), jnp.float32) for _ in range(NBUF)],
        [pltpu.SemaphoreType.DMA for _ in range(NBUF)],
        [pltpu.SemaphoreType.DMA for _ in range(NBUF)],
        pltpu.VMEM_SHARED((NACC, DH), jnp.float32),
    ],
    compiler_params=pltpu.CompilerParams(use_tc_tiling_on_sc=False),
)
def _sc_scatter(reff_hbm, col_hbm, u_hbm, z_hbm, out_hbm,
                reff_v, col_v, bufs, gsems, ssems, acc_sp):
    _scat_body(reff_hbm, col_hbm, u_hbm, z_hbm, out_hbm,
               reff_v, col_v, bufs, gsems, ssems, acc_sp)


def _tc_a_body(x_ref, w_ref, b_ref, d0_ref, d1_ref, h_ref, u_ref, dis_ref):
    h = jnp.tanh(jnp.dot(x_ref[...], w_ref[...],
                         preferred_element_type=jnp.float32) + b_ref[...])
    d = d0_ref[...] + d1_ref[...]
    dis = jnp.where(d > 0, lax.rsqrt(jnp.maximum(d, 1e-12)), 0.0)
    h_ref[...] = h
    dis_ref[...] = dis
    u = dis * h
    u_ref[0] = u[:, :DH]
    u_ref[1] = u[:, DH:]


def _tc_a(x, W_in, b_in, deg0, deg1):
    return pl.pallas_call(
        _tc_a_body,
        grid=(GRIDP,),
        in_specs=[
            pl.BlockSpec((BLKP, D), lambda i: (i, 0)),
            pl.BlockSpec((D, D), lambda i: (0, 0)),
            pl.BlockSpec((1, D), lambda i: (0, 0)),
            pl.BlockSpec((BLKP, 1), lambda i: (i, 0)),
            pl.BlockSpec((BLKP, 1), lambda i: (i, 0)),
        ],
        out_specs=[
            pl.BlockSpec((BLKP, D), lambda i: (i, 0)),
            pl.BlockSpec((2, BLKP, DH), lambda i: (0, i, 0)),
            pl.BlockSpec((BLKP, 1), lambda i: (i, 0)),
        ],
        out_shape=[
            jax.ShapeDtypeStruct((NACC, D), jnp.float32),
            jax.ShapeDtypeStruct((2, NACC, DH), jnp.float32),
            jax.ShapeDtypeStruct((NACC, 1), jnp.float32),
        ],
    )(x, W_in, b_in, deg0, deg1)


def _tc_b_body(s_ref, dis_ref, tx1_ref, u2_ref):
    dis = dis_ref[...]
    ss = jnp.concatenate([s_ref[0, 0] + s_ref[1, 0],
                          s_ref[0, 1] + s_ref[1, 1]], axis=1)
    tx1 = -dis * ss
    tx1_ref[...] = tx1
    u2 = dis * tx1
    u2_ref[0] = u2[:, :DH]
    u2_ref[1] = u2[:, DH:]


def _tc_b(s1, dis):
    return pl.pallas_call(
        _tc_b_body,
        grid=(GRIDP,),
        in_specs=[
            pl.BlockSpec((NCORES, 2, BLKP, DH), lambda i: (0, 0, i, 0)),
            pl.BlockSpec((BLKP, 1), lambda i: (i, 0)),
        ],
        out_specs=[
            pl.BlockSpec((BLKP, D), lambda i: (i, 0)),
            pl.BlockSpec((2, BLKP, DH), lambda i: (0, i, 0)),
        ],
        out_shape=[
            jax.ShapeDtypeStruct((NACC, D), jnp.float32),
            jax.ShapeDtypeStruct((2, NACC, DH), jnp.float32),
        ],
    )(s1, dis)


def _tc_c_body(h_ref, tx1_ref, s_ref, dis_ref, wc_ref, bc_ref, wo_ref,
               bo_ref, y_ref):
    h = h_ref[...]
    tx1 = tx1_ref[...]
    ss = jnp.concatenate([s_ref[0, 0] + s_ref[1, 0],
                          s_ref[0, 1] + s_ref[1, 1]], axis=1)
    tx2 = -2.0 * dis_ref[...] * ss - h
    out = (jnp.dot(h, wc_ref[0], preferred_element_type=jnp.float32)
           + jnp.dot(tx1, wc_ref[1], preferred_element_type=jnp.float32)
           + jnp.dot(tx2, wc_ref[2], preferred_element_type=jnp.float32)
           + bc_ref[...])
    y_ref[...] = jnp.dot(jnp.maximum(out, 0.0), wo_ref[...],
                         preferred_element_type=jnp.float32) + bo_ref[...]


def _tc_c(h, tx1, s2, dis, W_cheb, b_cheb, W_out, b_out):
    return pl.pallas_call(
        _tc_c_body,
        grid=(GRID,),
        in_specs=[
            pl.BlockSpec((BLK, D), lambda i: (i, 0)),
            pl.BlockSpec((BLK, D), lambda i: (i, 0)),
            pl.BlockSpec((NCORES, 2, BLK, DH), lambda i: (0, 0, i, 0)),
            pl.BlockSpec((BLK, 1), lambda i: (i, 0)),
            pl.BlockSpec((3, D, D), lambda i: (0, 0, 0)),
            pl.BlockSpec((1, D), lambda i: (0, 0)),
            pl.BlockSpec((D, C), lambda i: (0, 0)),
            pl.BlockSpec((1, C), lambda i: (0, 0)),
        ],
        out_specs=pl.BlockSpec((BLK, C), lambda i: (i, 0)),
        out_shape=jax.ShapeDtypeStruct((N, C), jnp.float32),
    )(h, tx1, s2, dis, W_cheb, b_cheb, W_out, b_out)


def kernel(x, edge_index, W_in, b_in, W_cheb, b_cheb, W_out, b_out):
    npad_e = NTILES * EPT - E
    k = jnp.arange(npad_e, dtype=jnp.int32)
    epad = jnp.stack([N + (k + 50) % (NACC - N), N + k % (NACC - N)])
    ei = jnp.concatenate([edge_index, epad], axis=1)
    row = ei[0].reshape(NTILES, NCH, CH)
    col = ei[1].reshape(NTILES, NCH, CH)
    zdeg = jnp.zeros((DEGN,), jnp.float32)
    znd = jnp.zeros((NACC, DH), jnp.float32)

    xp = jnp.concatenate([x, jnp.zeros((NACC - N, D), jnp.float32)], axis=0)
    deg0, deg1, roweff = _sc_deg(row, col, zdeg)
    h, u1, dis = _tc_a(xp, W_in, b_in.reshape(1, D),
                       deg0[:NACC].reshape(NACC, 1),
                       deg1[:NACC].reshape(NACC, 1))
    s1 = _sc_scatter(roweff, col, u1, znd)
    tx1, u2 = _tc_b(s1, dis)
    s2 = _sc_scatter(roweff, col, u2, znd)
    return _tc_c(h, tx1, s2, dis, W_cheb, b_cheb.reshape(1, D), W_out,
                 b_out.reshape(1, C))


# speedup vs baseline: 1.0044x; 1.0025x over previous
import functools

import jax
import jax.numpy as jnp
from jax import lax
from jax.experimental import pallas as pl
from jax.experimental.pallas import tpu as pltpu
from jax.experimental.pallas import tpu_sc as plsc

N = 10000
E = 320000
D = 128
C = 40
NCORES = 2
NSUB = 16
NTILES = NCORES * NSUB
CH = 128
NCH = 80
EPT = NCH * CH
NACC = 10112
RPT = NACC // NSUB
NPAD = NACC
DEGN = NSUB * 640
BLK = 1000
GRID = N // BLK
BLKP = NACC // 8
GRIDP = 8

_mesh = plsc.VectorSubcoreMesh(core_axis_name="c", subcore_axis_name="s")


def _deg_body(row_hbm, col_hbm, zdeg_hbm, deg0_out, deg1_out, roweff_out,
              row_v, col_v, eff_v, ones_v, deg_sp):
    c = lax.axis_index("c")
    s = lax.axis_index("s")
    wid = c * NSUB + s
    pltpu.sync_copy(zdeg_hbm.at[pl.ds(s * 640, 640)],
                    deg_sp.at[pl.ds(s * 640, 640)])
    for m in range(CH // 16):
        ones_v[pl.ds(m * 16, 16)] = jnp.ones((16,), jnp.float32)
    pltpu.sync_copy(row_hbm.at[wid], row_v)
    pltpu.sync_copy(col_hbm.at[wid], col_v)
    plsc.subcore_barrier()

    def chunk(j, carry):
        for m in range(CH // 16):
            r = row_v[j, pl.ds(m * 16, 16)]
            cc = col_v[j, pl.ds(m * 16, 16)]
            eff_v[j, pl.ds(m * 16, 16)] = jnp.where(r == cc, N, r)
        pltpu.sync_copy(ones_v, deg_sp.at[eff_v.at[j]], add=True)
        return carry

    lax.fori_loop(0, NCH, chunk, 0)
    pltpu.sync_copy(eff_v, roweff_out.at[wid])
    plsc.subcore_barrier()

    @pl.when(jnp.logical_and(s < 8, c == 0))
    def _():
        pltpu.sync_copy(deg_sp.at[pl.ds(s * 1280, 1280)],
                        deg0_out.at[pl.ds(s * 1280, 1280)])

    @pl.when(jnp.logical_and(s < 8, c == 1))
    def _():
        pltpu.sync_copy(deg_sp.at[pl.ds(s * 1280, 1280)],
                        deg1_out.at[pl.ds(s * 1280, 1280)])


@functools.partial(
    pl.kernel,
    out_type=(jax.ShapeDtypeStruct((DEGN,), jnp.float32),
              jax.ShapeDtypeStruct((DEGN,), jnp.float32),
              jax.ShapeDtypeStruct((NTILES, NCH, CH), jnp.int32)),
    mesh=_mesh,
    scratch_types=[
        pltpu.VMEM((NCH, CH), jnp.int32),
        pltpu.VMEM((NCH, CH), jnp.int32),
        pltpu.VMEM((NCH, CH), jnp.int32),
        pltpu.VMEM((CH,), jnp.float32),
        pltpu.VMEM_SHARED((DEGN,), jnp.float32),
    ],
)
def _sc_deg(row_hbm, col_hbm, zdeg_hbm, deg0_out, deg1_out, roweff_out,
            row_v, col_v, eff_v, ones_v, deg_sp):
    _deg_body(row_hbm, col_hbm, zdeg_hbm, deg0_out, deg1_out, roweff_out,
              row_v, col_v, eff_v, ones_v, deg_sp)


DH = D // 2


NBUF = 8
PFD = 3


def _scat_body(reff_hbm, col_hbm, u_hbm, z_hbm, out_hbm,
               reff_v, col_v, bufs, gsems, ssems, acc_sp):
    c = lax.axis_index("c")
    s = lax.axis_index("s")
    wid = c * NSUB + s
    pltpu.sync_copy(reff_hbm.at[wid], reff_v)
    pltpu.sync_copy(col_hbm.at[wid], col_v)

    for ph in range(2):
        pltpu.sync_copy(z_hbm.at[pl.ds(s * RPT, RPT)],
                        acc_sp.at[pl.ds(s * RPT, RPT)])
        plsc.subcore_barrier()
        u_ph = u_hbm.at[ph]

        def start_gather(j, b):
            pltpu.make_async_copy(u_ph.at[reff_v.at[j]], bufs[b],
                                  gsems[b]).start()

        def visit(j, b, prefetch):
            pltpu.make_async_copy(u_ph.at[reff_v.at[j]], bufs[b],
                                  gsems[b]).wait()
            if prefetch:
                start_gather(j + PFD, (b + PFD) % NBUF)
            pltpu.sync_copy(bufs[b], acc_sp.at[col_v.at[j]], add=True)

        for b in range(PFD):
            start_gather(b, b)

        def group(i, carry):
            for b in range(NBUF):
                j = i * NBUF + b
                visit(j, b, True)
            return carry

        def tail(_, carry):
            base = NCH - NBUF
            for b in range(NBUF):
                j = base + b
                visit(j, b, b + PFD < NBUF)
            return carry

        lax.fori_loop(0, NCH // NBUF - 1, group, 0)
        tail(0, 0)
        plsc.subcore_barrier()
        pltpu.sync_copy(acc_sp.at[pl.ds(s * RPT, RPT)],
                        out_hbm.at[c, ph, pl.ds(s * RPT, RPT)])


@functools.partial(
    pl.kernel,
    out_type=jax.ShapeDtypeStruct((NCORES, 2, NACC, DH), jnp.float32),
    mesh=_mesh,
    scratch_types=[
        pltpu.VMEM((NCH, CH), jnp.int32),
        pltpu.VMEM((NCH, CH), jnp.int32),
        [pltpu.VMEM((CH, DH), jnp.float32) for _ in range(NBUF)],
        [pltpu.SemaphoreType.DMA for _ in range(NBUF)],
        [pltpu.SemaphoreType.DMA for _ in range(NBUF)],
        pltpu.VMEM_SHARED((NACC, DH), jnp.float32),
    ],
    compiler_params=pltpu.CompilerParams(use_tc_tiling_on_sc=False),
)
def _sc_scatter(reff_hbm, col_hbm, u_hbm, z_hbm, out_hbm,
                reff_v, col_v, bufs, gsems, ssems, acc_sp):
    _scat_body(reff_hbm, col_hbm, u_hbm, z_hbm, out_hbm,
               reff_v, col_v, bufs, gsems, ssems, acc_sp)


def _tc_a_body(x_ref, w_ref, b_ref, d0_ref, d1_ref, h_ref, u_ref, dis_ref):
    h = jnp.tanh(jnp.dot(x_ref[...], w_ref[...],
                         preferred_element_type=jnp.float32) + b_ref[...])
    d = d0_ref[...] + d1_ref[...]
    dis = jnp.where(d > 0, lax.rsqrt(jnp.maximum(d, 1e-12)), 0.0)
    h_ref[...] = h
    dis_ref[...] = dis
    u = dis * h
    u_ref[0] = u[:, :DH]
    u_ref[1] = u[:, DH:]


def _tc_a(x, W_in, b_in, deg0, deg1):
    return pl.pallas_call(
        _tc_a_body,
        grid=(GRIDP,),
        in_specs=[
            pl.BlockSpec((BLKP, D), lambda i: (i, 0)),
            pl.BlockSpec((D, D), lambda i: (0, 0)),
            pl.BlockSpec((1, D), lambda i: (0, 0)),
            pl.BlockSpec((BLKP, 1), lambda i: (i, 0)),
            pl.BlockSpec((BLKP, 1), lambda i: (i, 0)),
        ],
        out_specs=[
            pl.BlockSpec((BLKP, D), lambda i: (i, 0)),
            pl.BlockSpec((2, BLKP, DH), lambda i: (0, i, 0)),
            pl.BlockSpec((BLKP, 1), lambda i: (i, 0)),
        ],
        out_shape=[
            jax.ShapeDtypeStruct((NACC, D), jnp.float32),
            jax.ShapeDtypeStruct((2, NACC, DH), jnp.float32),
            jax.ShapeDtypeStruct((NACC, 1), jnp.float32),
        ],
    )(x, W_in, b_in, deg0, deg1)


def _tc_b_body(s_ref, dis_ref, tx1_ref, u2_ref):
    dis = dis_ref[...]
    ss = jnp.concatenate([s_ref[0, 0] + s_ref[1, 0],
                          s_ref[0, 1] + s_ref[1, 1]], axis=1)
    tx1 = -dis * ss
    tx1_ref[...] = tx1
    u2 = dis * tx1
    u2_ref[0] = u2[:, :DH]
    u2_ref[1] = u2[:, DH:]


def _tc_b(s1, dis):
    return pl.pallas_call(
        _tc_b_body,
        grid=(GRIDP,),
        in_specs=[
            pl.BlockSpec((NCORES, 2, BLKP, DH), lambda i: (0, 0, i, 0)),
            pl.BlockSpec((BLKP, 1), lambda i: (i, 0)),
        ],
        out_specs=[
            pl.BlockSpec((BLKP, D), lambda i: (i, 0)),
            pl.BlockSpec((2, BLKP, DH), lambda i: (0, i, 0)),
        ],
        out_shape=[
            jax.ShapeDtypeStruct((NACC, D), jnp.float32),
            jax.ShapeDtypeStruct((2, NACC, DH), jnp.float32),
        ],
    )(s1, dis)


def _tc_c_body(h_ref, tx1_ref, s_ref, dis_ref, wc_ref, bc_ref, wo_ref,
               bo_ref, y_ref):
    h = h_ref[...]
    tx1 = tx1_ref[...]
    ss = jnp.concatenate([s_ref[0, 0] + s_ref[1, 0],
                          s_ref[0, 1] + s_ref[1, 1]], axis=1)
    tx2 = -2.0 * dis_ref[...] * ss - h
    out = (jnp.dot(h, wc_ref[0], preferred_element_type=jnp.float32)
           + jnp.dot(tx1, wc_ref[1], preferred_element_type=jnp.float32)
           + jnp.dot(tx2, wc_ref[2], preferred_element_type=jnp.float32)
           + bc_ref[...])
    y_ref[...] = jnp.dot(jnp.maximum(out, 0.0), wo_ref[...],
                         preferred_element_type=jnp.float32) + bo_ref[...]


def _tc_c(h, tx1, s2, dis, W_cheb, b_cheb, W_out, b_out):
    return pl.pallas_call(
        _tc_c_body,
        grid=(GRID,),
        in_specs=[
            pl.BlockSpec((BLK, D), lambda i: (i, 0)),
            pl.BlockSpec((BLK, D), lambda i: (i, 0)),
            pl.BlockSpec((NCORES, 2, BLK, DH), lambda i: (0, 0, i, 0)),
            pl.BlockSpec((BLK, 1), lambda i: (i, 0)),
            pl.BlockSpec((3, D, D), lambda i: (0, 0, 0)),
            pl.BlockSpec((1, D), lambda i: (0, 0)),
            pl.BlockSpec((D, C), lambda i: (0, 0)),
            pl.BlockSpec((1, C), lambda i: (0, 0)),
        ],
        out_specs=pl.BlockSpec((BLK, C), lambda i: (i, 0)),
        out_shape=jax.ShapeDtypeStruct((N, C), jnp.float32),
    )(h, tx1, s2, dis, W_cheb, b_cheb, W_out, b_out)


def kernel(x, edge_index, W_in, b_in, W_cheb, b_cheb, W_out, b_out):
    npad_e = NTILES * EPT - E
    k = jnp.arange(npad_e, dtype=jnp.int32)
    epad = jnp.stack([N + (k + 50) % (NACC - N), N + k % (NACC - N)])
    ei = jnp.concatenate([edge_index, epad], axis=1)
    row = ei[0].reshape(NTILES, NCH, CH)
    col = ei[1].reshape(NTILES, NCH, CH)
    zdeg = jnp.zeros((DEGN,), jnp.float32)
    znd = jnp.zeros((NACC, DH), jnp.float32)

    xp = jnp.concatenate([x, jnp.zeros((NACC - N, D), jnp.float32)], axis=0)
    deg0, deg1, roweff = _sc_deg(row, col, zdeg)
    h, u1, dis = _tc_a(xp, W_in, b_in.reshape(1, D),
                       deg0[:NACC].reshape(NACC, 1),
                       deg1[:NACC].reshape(NACC, 1))
    s1 = _sc_scatter(roweff, col, u1, znd)
    tx1, u2 = _tc_b(s1, dis)
    s2 = _sc_scatter(roweff, col, u2, znd)
    return _tc_c(h, tx1, s2, dis, W_cheb, b_cheb.reshape(1, D), W_out,
                 b_out.reshape(1, C))


# speedup vs baseline: 1.0050x; 1.0006x over previous
import functools

import jax
import jax.numpy as jnp
from jax import lax
from jax.experimental import pallas as pl
from jax.experimental.pallas import tpu as pltpu
from jax.experimental.pallas import tpu_sc as plsc

N = 10000
E = 320000
D = 128
C = 40
NCORES = 2
NSUB = 16
NTILES = NCORES * NSUB
CH = 128
NCH = 80
EPT = NCH * CH
NACC = 10112
RPT = NACC // NSUB
NPAD = NACC
DEGN = NSUB * 640
BLK = 1000
GRID = N // BLK
BLKP = NACC // 8
GRIDP = 8

_mesh = plsc.VectorSubcoreMesh(core_axis_name="c", subcore_axis_name="s")


def _deg_body(row_hbm, col_hbm, zdeg_hbm, deg0_out, deg1_out, roweff_out,
              row_v, col_v, eff_v, ones_v, deg_sp):
    c = lax.axis_index("c")
    s = lax.axis_index("s")
    wid = c * NSUB + s
    pltpu.sync_copy(zdeg_hbm.at[pl.ds(s * 640, 640)],
                    deg_sp.at[pl.ds(s * 640, 640)])
    for m in range(CH // 16):
        ones_v[pl.ds(m * 16, 16)] = jnp.ones((16,), jnp.float32)
    pltpu.sync_copy(row_hbm.at[wid], row_v)
    pltpu.sync_copy(col_hbm.at[wid], col_v)
    plsc.subcore_barrier()

    def chunk(j, carry):
        for m in range(CH // 16):
            r = row_v[j, pl.ds(m * 16, 16)]
            cc = col_v[j, pl.ds(m * 16, 16)]
            eff_v[j, pl.ds(m * 16, 16)] = jnp.where(r == cc, N, r)
        pltpu.sync_copy(ones_v, deg_sp.at[eff_v.at[j]], add=True)
        return carry

    lax.fori_loop(0, NCH, chunk, 0)
    pltpu.sync_copy(eff_v, roweff_out.at[wid])
    plsc.subcore_barrier()

    @pl.when(jnp.logical_and(s < 8, c == 0))
    def _():
        pltpu.sync_copy(deg_sp.at[pl.ds(s * 1280, 1280)],
                        deg0_out.at[pl.ds(s * 1280, 1280)])

    @pl.when(jnp.logical_and(s < 8, c == 1))
    def _():
        pltpu.sync_copy(deg_sp.at[pl.ds(s * 1280, 1280)],
                        deg1_out.at[pl.ds(s * 1280, 1280)])


@functools.partial(
    pl.kernel,
    out_type=(jax.ShapeDtypeStruct((DEGN,), jnp.float32),
              jax.ShapeDtypeStruct((DEGN,), jnp.float32),
              jax.ShapeDtypeStruct((NTILES, NCH, CH), jnp.int32)),
    mesh=_mesh,
    scratch_types=[
        pltpu.VMEM((NCH, CH), jnp.int32),
        pltpu.VMEM((NCH, CH), jnp.int32),
        pltpu.VMEM((NCH, CH), jnp.int32),
        pltpu.VMEM((CH,), jnp.float32),
        pltpu.VMEM_SHARED((DEGN,), jnp.float32),
    ],
    compiler_params=pltpu.CompilerParams(use_tc_tiling_on_sc=False),
)
def _sc_deg(row_hbm, col_hbm, zdeg_hbm, deg0_out, deg1_out, roweff_out,
            row_v, col_v, eff_v, ones_v, deg_sp):
    _deg_body(row_hbm, col_hbm, zdeg_hbm, deg0_out, deg1_out, roweff_out,
              row_v, col_v, eff_v, ones_v, deg_sp)


DH = D // 2


NBUF = 8
PFD = 3


def _scat_body(reff_hbm, col_hbm, u_hbm, z_hbm, out_hbm,
               reff_v, col_v, bufs, gsems, ssems, acc_sp):
    c = lax.axis_index("c")
    s = lax.axis_index("s")
    wid = c * NSUB + s
    pltpu.sync_copy(reff_hbm.at[wid], reff_v)
    pltpu.sync_copy(col_hbm.at[wid], col_v)

    for ph in range(2):
        pltpu.sync_copy(z_hbm.at[pl.ds(s * RPT, RPT)],
                        acc_sp.at[pl.ds(s * RPT, RPT)])
        plsc.subcore_barrier()
        u_ph = u_hbm.at[ph]

        def start_gather(j, b):
            pltpu.make_async_copy(u_ph.at[reff_v.at[j]], bufs[b],
                                  gsems[b]).start()

        def visit(j, b, prefetch):
            pltpu.make_async_copy(u_ph.at[reff_v.at[j]], bufs[b],
                                  gsems[b]).wait()
            if prefetch:
                start_gather(j + PFD, (b + PFD) % NBUF)
            pltpu.sync_copy(bufs[b], acc_sp.at[col_v.at[j]], add=True)

        for b in range(PFD):
            start_gather(b, b)

        def group(i, carry):
            for b in range(NBUF):
                j = i * NBUF + b
                visit(j, b, True)
            return carry

        def tail(_, carry):
            base = NCH - NBUF
            for b in range(NBUF):
                j = base + b
                visit(j, b, b + PFD < NBUF)
            return carry

        lax.fori_loop(0, NCH // NBUF - 1, group, 0)
        tail(0, 0)
        plsc.subcore_barrier()
        pltpu.sync_copy(acc_sp.at[pl.ds(s * RPT, RPT)],
                        out_hbm.at[c, ph, pl.ds(s * RPT, RPT)])


@functools.partial(
    pl.kernel,
    out_type=jax.ShapeDtypeStruct((NCORES, 2, NACC, DH), jnp.float32),
    mesh=_mesh,
    scratch_types=[
        pltpu.VMEM((NCH, CH), jnp.int32),
        pltpu.VMEM((NCH, CH), jnp.int32),
        [pltpu.VMEM((CH, DH), jnp.float32) for _ in range(NBUF)],
        [pltpu.SemaphoreType.DMA for _ in range(NBUF)],
        [pltpu.SemaphoreType.DMA for _ in range(NBUF)],
        pltpu.VMEM_SHARED((NACC, DH), jnp.float32),
    ],
    compiler_params=pltpu.CompilerParams(use_tc_tiling_on_sc=False),
)
def _sc_scatter(reff_hbm, col_hbm, u_hbm, z_hbm, out_hbm,
                reff_v, col_v, bufs, gsems, ssems, acc_sp):
    _scat_body(reff_hbm, col_hbm, u_hbm, z_hbm, out_hbm,
               reff_v, col_v, bufs, gsems, ssems, acc_sp)


def _tc_a_body(x_ref, w_ref, b_ref, d0_ref, d1_ref, h_ref, u_ref, dis_ref):
    h = jnp.tanh(jnp.dot(x_ref[...], w_ref[...],
                         preferred_element_type=jnp.float32) + b_ref[...])
    d = d0_ref[...] + d1_ref[...]
    dis = jnp.where(d > 0, lax.rsqrt(jnp.maximum(d, 1e-12)), 0.0)
    h_ref[...] = h
    dis_ref[...] = dis
    u = dis * h
    u_ref[0] = u[:, :DH]
    u_ref[1] = u[:, DH:]


def _tc_a(x, W_in, b_in, deg0, deg1):
    return pl.pallas_call(
        _tc_a_body,
        grid=(GRIDP,),
        in_specs=[
            pl.BlockSpec((BLKP, D), lambda i: (i, 0)),
            pl.BlockSpec((D, D), lambda i: (0, 0)),
            pl.BlockSpec((1, D), lambda i: (0, 0)),
            pl.BlockSpec((BLKP, 1), lambda i: (i, 0)),
            pl.BlockSpec((BLKP, 1), lambda i: (i, 0)),
        ],
        out_specs=[
            pl.BlockSpec((BLKP, D), lambda i: (i, 0)),
            pl.BlockSpec((2, BLKP, DH), lambda i: (0, i, 0)),
            pl.BlockSpec((BLKP, 1), lambda i: (i, 0)),
        ],
        out_shape=[
            jax.ShapeDtypeStruct((NACC, D), jnp.float32),
            jax.ShapeDtypeStruct((2, NACC, DH), jnp.float32),
            jax.ShapeDtypeStruct((NACC, 1), jnp.float32),
        ],
    )(x, W_in, b_in, deg0, deg1)


def _tc_b_body(s_ref, dis_ref, tx1_ref, u2_ref):
    dis = dis_ref[...]
    ss = jnp.concatenate([s_ref[0, 0] + s_ref[1, 0],
                          s_ref[0, 1] + s_ref[1, 1]], axis=1)
    tx1 = -dis * ss
    tx1_ref[...] = tx1
    u2 = dis * tx1
    u2_ref[0] = u2[:, :DH]
    u2_ref[1] = u2[:, DH:]


def _tc_b(s1, dis):
    return pl.pallas_call(
        _tc_b_body,
        grid=(GRIDP,),
        in_specs=[
            pl.BlockSpec((NCORES, 2, BLKP, DH), lambda i: (0, 0, i, 0)),
            pl.BlockSpec((BLKP, 1), lambda i: (i, 0)),
        ],
        out_specs=[
            pl.BlockSpec((BLKP, D), lambda i: (i, 0)),
            pl.BlockSpec((2, BLKP, DH), lambda i: (0, i, 0)),
        ],
        out_shape=[
            jax.ShapeDtypeStruct((NACC, D), jnp.float32),
            jax.ShapeDtypeStruct((2, NACC, DH), jnp.float32),
        ],
    )(s1, dis)


def _tc_c_body(h_ref, tx1_ref, s_ref, dis_ref, wc_ref, bc_ref, wo_ref,
               bo_ref, y_ref):
    h = h_ref[...]
    tx1 = tx1_ref[...]
    ss = jnp.concatenate([s_ref[0, 0] + s_ref[1, 0],
                          s_ref[0, 1] + s_ref[1, 1]], axis=1)
    tx2 = -2.0 * dis_ref[...] * ss - h
    out = (jnp.dot(h, wc_ref[0], preferred_element_type=jnp.float32)
           + jnp.dot(tx1, wc_ref[1], preferred_element_type=jnp.float32)
           + jnp.dot(tx2, wc_ref[2], preferred_element_type=jnp.float32)
           + bc_ref[...])
    y_ref[...] = jnp.dot(jnp.maximum(out, 0.0), wo_ref[...],
                         preferred_element_type=jnp.float32) + bo_ref[...]


def _tc_c(h, tx1, s2, dis, W_cheb, b_cheb, W_out, b_out):
    return pl.pallas_call(
        _tc_c_body,
        grid=(GRID,),
        in_specs=[
            pl.BlockSpec((BLK, D), lambda i: (i, 0)),
            pl.BlockSpec((BLK, D), lambda i: (i, 0)),
            pl.BlockSpec((NCORES, 2, BLK, DH), lambda i: (0, 0, i, 0)),
            pl.BlockSpec((BLK, 1), lambda i: (i, 0)),
            pl.BlockSpec((3, D, D), lambda i: (0, 0, 0)),
            pl.BlockSpec((1, D), lambda i: (0, 0)),
            pl.BlockSpec((D, C), lambda i: (0, 0)),
            pl.BlockSpec((1, C), lambda i: (0, 0)),
        ],
        out_specs=pl.BlockSpec((BLK, C), lambda i: (i, 0)),
        out_shape=jax.ShapeDtypeStruct((N, C), jnp.float32),
    )(h, tx1, s2, dis, W_cheb, b_cheb, W_out, b_out)


def kernel(x, edge_index, W_in, b_in, W_cheb, b_cheb, W_out, b_out):
    npad_e = NTILES * EPT - E
    k = jnp.arange(npad_e, dtype=jnp.int32)
    epad = jnp.stack([N + (k + 50) % (NACC - N), N + k % (NACC - N)])
    ei = jnp.concatenate([edge_index, epad], axis=1)
    row = ei[0].reshape(NTILES, NCH, CH)
    col = ei[1].reshape(NTILES, NCH, CH)
    zdeg = jnp.zeros((DEGN,), jnp.float32)
    znd = jnp.zeros((NACC, DH), jnp.float32)

    xp = jnp.concatenate([x, jnp.zeros((NACC - N, D), jnp.float32)], axis=0)
    deg0, deg1, roweff = _sc_deg(row, col, zdeg)
    h, u1, dis = _tc_a(xp, W_in, b_in.reshape(1, D),
                       deg0[:NACC].reshape(NACC, 1),
                       deg1[:NACC].reshape(NACC, 1))
    s1 = _sc_scatter(roweff, col, u1, znd)
    tx1, u2 = _tc_b(s1, dis)
    s2 = _sc_scatter(roweff, col, u2, znd)
    return _tc_c(h, tx1, s2, dis, W_cheb, b_cheb.reshape(1, D), W_out,
                 b_out.reshape(1, C))


# speedup vs baseline: 1.0119x; 1.0069x over previous
import functools

import jax
import jax.numpy as jnp
from jax import lax
from jax.experimental import pallas as pl
from jax.experimental.pallas import tpu as pltpu
from jax.experimental.pallas import tpu_sc as plsc

N = 10000
E = 320000
D = 128
C = 40
NCORES = 2
NSUB = 16
NTILES = NCORES * NSUB
CH = 128
NCH = 80
EPT = NCH * CH
NACC = 10112
RPT = NACC // NSUB
NPAD = NACC
DEGN = NSUB * 640
BLK = 1000
GRID = N // BLK
BLKP = NACC // 8
GRIDP = 8

_mesh = plsc.VectorSubcoreMesh(core_axis_name="c", subcore_axis_name="s")


def _deg_body(row_hbm, col_hbm, zdeg_hbm, deg0_out, deg1_out, roweff_out,
              row_v, col_v, eff_v, ones_v, deg_sp):
    c = lax.axis_index("c")
    s = lax.axis_index("s")
    wid = c * NSUB + s
    pltpu.sync_copy(zdeg_hbm.at[pl.ds(s * 640, 640)],
                    deg_sp.at[pl.ds(s * 640, 640)])
    for m in range(CH // 16):
        ones_v[pl.ds(m * 16, 16)] = jnp.ones((16,), jnp.float32)
    pltpu.sync_copy(row_hbm.at[wid], row_v)
    pltpu.sync_copy(col_hbm.at[wid], col_v)
    plsc.subcore_barrier()

    def chunk(j, carry):
        for m in range(CH // 16):
            r = row_v[j, pl.ds(m * 16, 16)]
            cc = col_v[j, pl.ds(m * 16, 16)]
            eff_v[j, pl.ds(m * 16, 16)] = jnp.where(r == cc, N, r)
        pltpu.sync_copy(ones_v, deg_sp.at[eff_v.at[j]], add=True)
        return carry

    lax.fori_loop(0, NCH, chunk, 0)
    pltpu.sync_copy(eff_v, roweff_out.at[wid])
    plsc.subcore_barrier()

    @pl.when(jnp.logical_and(s < 8, c == 0))
    def _():
        pltpu.sync_copy(deg_sp.at[pl.ds(s * 1280, 1280)],
                        deg0_out.at[pl.ds(s * 1280, 1280)])

    @pl.when(jnp.logical_and(s < 8, c == 1))
    def _():
        pltpu.sync_copy(deg_sp.at[pl.ds(s * 1280, 1280)],
                        deg1_out.at[pl.ds(s * 1280, 1280)])


@functools.partial(
    pl.kernel,
    out_type=(jax.ShapeDtypeStruct((DEGN,), jnp.float32),
              jax.ShapeDtypeStruct((DEGN,), jnp.float32),
              jax.ShapeDtypeStruct((NTILES, NCH, CH), jnp.int32)),
    mesh=_mesh,
    scratch_types=[
        pltpu.VMEM((NCH, CH), jnp.int32),
        pltpu.VMEM((NCH, CH), jnp.int32),
        pltpu.VMEM((NCH, CH), jnp.int32),
        pltpu.VMEM((CH,), jnp.float32),
        pltpu.VMEM_SHARED((DEGN,), jnp.float32),
    ],
    compiler_params=pltpu.CompilerParams(use_tc_tiling_on_sc=False),
)
def _sc_deg(row_hbm, col_hbm, zdeg_hbm, deg0_out, deg1_out, roweff_out,
            row_v, col_v, eff_v, ones_v, deg_sp):
    _deg_body(row_hbm, col_hbm, zdeg_hbm, deg0_out, deg1_out, roweff_out,
              row_v, col_v, eff_v, ones_v, deg_sp)


DH = D // 2


NBUF = 8
PFD = 3


def _scat_body(reff_hbm, col_hbm, u_hbm, z_hbm, out_hbm,
               reff_v, col_v, bufs, gsems, ssems, acc_sp):
    c = lax.axis_index("c")
    s = lax.axis_index("s")
    wid = c * NSUB + s
    pltpu.sync_copy(reff_hbm.at[wid], reff_v)
    pltpu.sync_copy(col_hbm.at[wid], col_v)

    def zero_acc():
        pltpu.sync_copy(z_hbm.at[pl.ds(s * RPT, RPT)],
                        acc_sp.at[pl.ds(s * RPT, RPT)])

    def copy_out(ph):
        pltpu.sync_copy(acc_sp.at[pl.ds(s * RPT, RPT)],
                        out_hbm.at[c, ph, pl.ds(s * RPT, RPT)])

    def start_gather(u_ph, j, b):
        pltpu.make_async_copy(u_ph.at[reff_v.at[j]], bufs[b],
                              gsems[b]).start()

    def main_loop(ph):
        u_ph = u_hbm.at[ph]

        def visit(j, b, prefetch):
            pltpu.make_async_copy(u_ph.at[reff_v.at[j]], bufs[b],
                                  gsems[b]).wait()
            if prefetch:
                start_gather(u_ph, j + PFD, (b + PFD) % NBUF)
            pltpu.sync_copy(bufs[b], acc_sp.at[col_v.at[j]], add=True)

        def group(i, carry):
            for b in range(NBUF):
                j = i * NBUF + b
                visit(j, b, True)
            return carry

        def tail(_, carry):
            base = NCH - NBUF
            for b in range(NBUF):
                j = base + b
                visit(j, b, b + PFD < NBUF)
            return carry

        lax.fori_loop(0, NCH // NBUF - 1, group, 0)
        tail(0, 0)

    zero_acc()
    plsc.subcore_barrier()
    for b in range(PFD):
        start_gather(u_hbm.at[0], b, b)
    main_loop(0)
    plsc.subcore_barrier()
    for b in range(PFD):
        start_gather(u_hbm.at[1], b, b)
    copy_out(0)
    zero_acc()
    plsc.subcore_barrier()
    main_loop(1)
    plsc.subcore_barrier()
    copy_out(1)


@functools.partial(
    pl.kernel,
    out_type=jax.ShapeDtypeStruct((NCORES, 2, NACC, DH), jnp.float32),
    mesh=_mesh,
    scratch_types=[
        pltpu.VMEM((NCH, CH), jnp.int32),
        pltpu.VMEM((NCH, CH), jnp.int32),
        [pltpu.VMEM((CH, DH), jnp.float32) for _ in range(NBUF)],
        [pltpu.SemaphoreType.DMA for _ in range(NBUF)],
        [pltpu.SemaphoreType.DMA for _ in range(NBUF)],
        pltpu.VMEM_SHARED((NACC, DH), jnp.float32),
    ],
    compiler_params=pltpu.CompilerParams(use_tc_tiling_on_sc=False),
)
def _sc_scatter(reff_hbm, col_hbm, u_hbm, z_hbm, out_hbm,
                reff_v, col_v, bufs, gsems, ssems, acc_sp):
    _scat_body(reff_hbm, col_hbm, u_hbm, z_hbm, out_hbm,
               reff_v, col_v, bufs, gsems, ssems, acc_sp)


def _tc_a_body(x_ref, w_ref, b_ref, d0_ref, d1_ref, h_ref, u_ref, dis_ref):
    h = jnp.tanh(jnp.dot(x_ref[...], w_ref[...],
                         preferred_element_type=jnp.float32) + b_ref[...])
    d = d0_ref[...] + d1_ref[...]
    dis = jnp.where(d > 0, lax.rsqrt(jnp.maximum(d, 1e-12)), 0.0)
    h_ref[...] = h
    dis_ref[...] = dis
    u = dis * h
    u_ref[0] = u[:, :DH]
    u_ref[1] = u[:, DH:]


def _tc_a(x, W_in, b_in, deg0, deg1):
    return pl.pallas_call(
        _tc_a_body,
        grid=(GRIDP,),
        in_specs=[
            pl.BlockSpec((BLKP, D), lambda i: (i, 0)),
            pl.BlockSpec((D, D), lambda i: (0, 0)),
            pl.BlockSpec((1, D), lambda i: (0, 0)),
            pl.BlockSpec((BLKP, 1), lambda i: (i, 0)),
            pl.BlockSpec((BLKP, 1), lambda i: (i, 0)),
        ],
        out_specs=[
            pl.BlockSpec((BLKP, D), lambda i: (i, 0)),
            pl.BlockSpec((2, BLKP, DH), lambda i: (0, i, 0)),
            pl.BlockSpec((BLKP, 1), lambda i: (i, 0)),
        ],
        out_shape=[
            jax.ShapeDtypeStruct((NACC, D), jnp.float32),
            jax.ShapeDtypeStruct((2, NACC, DH), jnp.float32),
            jax.ShapeDtypeStruct((NACC, 1), jnp.float32),
        ],
    )(x, W_in, b_in, deg0, deg1)


def _tc_b_body(s_ref, dis_ref, tx1_ref, u2_ref):
    dis = dis_ref[...]
    ss = jnp.concatenate([s_ref[0, 0] + s_ref[1, 0],
                          s_ref[0, 1] + s_ref[1, 1]], axis=1)
    tx1 = -dis * ss
    tx1_ref[...] = tx1
    u2 = dis * tx1
    u2_ref[0] = u2[:, :DH]
    u2_ref[1] = u2[:, DH:]


def _tc_b(s1, dis):
    return pl.pallas_call(
        _tc_b_body,
        grid=(GRIDP,),
        in_specs=[
            pl.BlockSpec((NCORES, 2, BLKP, DH), lambda i: (0, 0, i, 0)),
            pl.BlockSpec((BLKP, 1), lambda i: (i, 0)),
        ],
        out_specs=[
            pl.BlockSpec((BLKP, D), lambda i: (i, 0)),
            pl.BlockSpec((2, BLKP, DH), lambda i: (0, i, 0)),
        ],
        out_shape=[
            jax.ShapeDtypeStruct((NACC, D), jnp.float32),
            jax.ShapeDtypeStruct((2, NACC, DH), jnp.float32),
        ],
    )(s1, dis)


def _tc_c_body(h_ref, tx1_ref, s_ref, dis_ref, wc_ref, bc_ref, wo_ref,
               bo_ref, y_ref):
    h = h_ref[...]
    tx1 = tx1_ref[...]
    ss = jnp.concatenate([s_ref[0, 0] + s_ref[1, 0],
                          s_ref[0, 1] + s_ref[1, 1]], axis=1)
    tx2 = -2.0 * dis_ref[...] * ss - h
    out = (jnp.dot(h, wc_ref[0], preferred_element_type=jnp.float32)
           + jnp.dot(tx1, wc_ref[1], preferred_element_type=jnp.float32)
           + jnp.dot(tx2, wc_ref[2], preferred_element_type=jnp.float32)
           + bc_ref[...])
    y_ref[...] = jnp.dot(jnp.maximum(out, 0.0), wo_ref[...],
                         preferred_element_type=jnp.float32) + bo_ref[...]


def _tc_c(h, tx1, s2, dis, W_cheb, b_cheb, W_out, b_out):
    return pl.pallas_call(
        _tc_c_body,
        grid=(GRID,),
        in_specs=[
            pl.BlockSpec((BLK, D), lambda i: (i, 0)),
            pl.BlockSpec((BLK, D), lambda i: (i, 0)),
            pl.BlockSpec((NCORES, 2, BLK, DH), lambda i: (0, 0, i, 0)),
            pl.BlockSpec((BLK, 1), lambda i: (i, 0)),
            pl.BlockSpec((3, D, D), lambda i: (0, 0, 0)),
            pl.BlockSpec((1, D), lambda i: (0, 0)),
            pl.BlockSpec((D, C), lambda i: (0, 0)),
            pl.BlockSpec((1, C), lambda i: (0, 0)),
        ],
        out_specs=pl.BlockSpec((BLK, C), lambda i: (i, 0)),
        out_shape=jax.ShapeDtypeStruct((N, C), jnp.float32),
    )(h, tx1, s2, dis, W_cheb, b_cheb, W_out, b_out)


def kernel(x, edge_index, W_in, b_in, W_cheb, b_cheb, W_out, b_out):
    npad_e = NTILES * EPT - E
    k = jnp.arange(npad_e, dtype=jnp.int32)
    epad = jnp.stack([N + (k + 50) % (NACC - N), N + k % (NACC - N)])
    ei = jnp.concatenate([edge_index, epad], axis=1)
    row = ei[0].reshape(NTILES, NCH, CH)
    col = ei[1].reshape(NTILES, NCH, CH)
    zdeg = jnp.zeros((DEGN,), jnp.float32)
    znd = jnp.zeros((NACC, DH), jnp.float32)

    xp = jnp.concatenate([x, jnp.zeros((NACC - N, D), jnp.float32)], axis=0)
    deg0, deg1, roweff = _sc_deg(row, col, zdeg)
    h, u1, dis = _tc_a(xp, W_in, b_in.reshape(1, D),
                       deg0[:NACC].reshape(NACC, 1),
                       deg1[:NACC].reshape(NACC, 1))
    s1 = _sc_scatter(roweff, col, u1, znd)
    tx1, u2 = _tc_b(s1, dis)
    s2 = _sc_scatter(roweff, col, u2, znd)
    return _tc_c(h, tx1, s2, dis, W_cheb, b_cheb.reshape(1, D), W_out,
                 b_out.reshape(1, C))
